# baseline (speedup 1.0000x reference)
import functools

import jax
import jax.numpy as jnp
from jax import lax
from jax.experimental import pallas as pl
from jax.experimental.pallas import tpu as pltpu
from jax.experimental.pallas import tpu_sc as plsc

N = 320000
NSEG = 10000
NSEG_PAD = 10240
D = 128
BN_EPS = 1e-5

NC = 2
NS = 16
NW = NC * NS
ROWS_PER_W = N // NW
CH = 80
NCH = ROWS_PER_W // CH
SEG_SLICE = NSEG_PAD // NS

_mesh = plsc.VectorSubcoreMesh(core_axis_name="c", subcore_axis_name="s")


@functools.partial(
    pl.kernel,
    out_type=jax.ShapeDtypeStruct((NC, NSEG_PAD, D), jnp.float32),
    mesh=_mesh,
    scratch_types=[
        pltpu.VMEM((NCH, CH), jnp.int32),
        pltpu.VMEM((CH, D), jnp.float32),
        pltpu.VMEM((CH, D), jnp.float32),
        pltpu.VMEM((CH, D), jnp.float32),
        pltpu.VMEM_SHARED((NSEG_PAD, D), jnp.float32),
        pltpu.SemaphoreType.DMA,
        pltpu.SemaphoreType.DMA,
        pltpu.SemaphoreType.DMA,
        pltpu.SemaphoreType.DMA,
        pltpu.SemaphoreType.DMA,
        pltpu.SemaphoreType.DMA,
    ],
)
def _segment_sum_sc(x_hbm, batch_hbm, zeros_hbm, out_hbm,
                    idx_v, x0, x1, x2, table_sh,
                    gs0, gs1, gs2, ss0, ss1, ss2):
    c = lax.axis_index("c")
    s = lax.axis_index("s")
    wid = c * NS + s
    base = wid * ROWS_PER_W
    bufs = (x0, x1, x2)
    gsems = (gs0, gs1, gs2)
    ssems = (ss0, ss1, ss2)

    def chunk(j):
        return x_hbm.at[pl.ds(base + j * CH, CH)]

    pltpu.sync_copy(batch_hbm.at[wid], idx_v)
    pltpu.sync_copy(zeros_hbm, table_sh.at[pl.ds(s * SEG_SLICE, SEG_SLICE)])
    plsc.subcore_barrier()

    pltpu.async_copy(chunk(0), bufs[0], gsems[0])
    pltpu.async_copy(chunk(1), bufs[1], gsems[1])

    def body(j3, carry):
        for k in range(3):
            j = 3 * j3 + k
            k2 = (k + 2) % 3
            pltpu.make_async_copy(chunk(j), bufs[k], gsems[k]).wait()
            pltpu.async_copy(bufs[k], table_sh.at[idx_v.at[j]], ssems[k],
                             add=True)
            if k == 0:
                @pl.when(j3 > 0)
                def _():
                    pltpu.make_async_copy(
                        bufs[k2], table_sh.at[idx_v.at[j]], ssems[k2]).wait()
            else:
                pltpu.make_async_copy(
                    bufs[k2], table_sh.at[idx_v.at[j]], ssems[k2]).wait()
            pltpu.async_copy(chunk(j + 2), bufs[k2], gsems[k2])
        return carry

    lax.fori_loop(0, (NCH - 2) // 3, body, 0)
    pltpu.make_async_copy(chunk(NCH - 2), bufs[0], gsems[0]).wait()
    pltpu.make_async_copy(bufs[2], table_sh.at[idx_v.at[NCH - 3]], ssems[2]).wait()
    sc123 = pltpu.async_copy(bufs[0], table_sh.at[idx_v.at[NCH - 2]], ssems[0],
                             add=True)
    pltpu.make_async_copy(chunk(NCH - 1), bufs[1], gsems[1]).wait()
    sc123.wait()
    pltpu.sync_copy(bufs[1], table_sh.at[idx_v.at[NCH - 1]], add=True)
    plsc.subcore_barrier()
    pltpu.sync_copy(
        table_sh.at[pl.ds(s * SEG_SLICE, SEG_SLICE)],
        out_hbm.at[c, pl.ds(s * SEG_SLICE, SEG_SLICE)],
    )


def _bn_body(p_ref, wlin_ref, gamma_ref, beta_ref, out_ref):
    summ = p_ref[0] + p_ref[1]
    summ = lax.dot_general(
        summ, wlin_ref[...], (((1,), (1,)), ((), ())),
        preferred_element_type=jnp.float32,
    )
    mean = jnp.sum(summ, axis=0, keepdims=True) / NSEG
    cent = summ - mean
    ssq = jnp.sum(cent * cent, axis=0, keepdims=True) - (
        (NSEG_PAD - NSEG) * mean * mean
    )
    var = ssq / NSEG
    y = cent / jnp.sqrt(var + BN_EPS) * gamma_ref[...] + beta_ref[...]
    out_ref[...] = jnp.maximum(y, 0.0).astype(jnp.bfloat16)


_bn_call = pl.pallas_call(
    _bn_body,
    out_shape=jax.ShapeDtypeStruct((NSEG_PAD, D), jnp.bfloat16),
)


DP = D // 2


@functools.partial(
    pl.kernel,
    out_type=jax.ShapeDtypeStruct((N, DP), jnp.float32),
    mesh=_mesh,
    scratch_types=[
        pltpu.VMEM((NCH, CH), jnp.int32),
        pltpu.VMEM((CH, DP), jnp.float32),
        pltpu.VMEM((CH, DP), jnp.float32),
        pltpu.VMEM_SHARED((NSEG_PAD, DP), jnp.float32),
        pltpu.SemaphoreType.DMA,
        pltpu.SemaphoreType.DMA,
        pltpu.SemaphoreType.DMA,
        pltpu.SemaphoreType.DMA,
    ],
)
def _gather_sc(table_hbm, batch_hbm, out_hbm,
               idx_v, ga, gb, table_sh, gsa, gsb, wsa, wsb):
    c = lax.axis_index("c")
    s = lax.axis_index("s")
    wid = c * NS + s
    base = wid * ROWS_PER_W

    def outref(j):
        return out_hbm.at[pl.ds(base + j * CH, CH)]

    pltpu.sync_copy(
        table_hbm.at[pl.ds(s * SEG_SLICE, SEG_SLICE)],
        table_sh.at[pl.ds(s * SEG_SLICE, SEG_SLICE)],
    )
    pltpu.sync_copy(batch_hbm.at[wid], idx_v)
    plsc.subcore_barrier()

    pltpu.async_copy(table_sh.at[idx_v.at[0]], ga, gsa)

    def body(j2, carry):
        j = 2 * j2
        pltpu.make_async_copy(table_sh.at[idx_v.at[j]], ga, gsa).wait()

        @pl.when(j2 > 0)
        def _():
            pltpu.make_async_copy(gb, outref(j - 1), wsb).wait()

        pltpu.async_copy(table_sh.at[idx_v.at[j + 1]], gb, gsb)
        wa = pltpu.async_copy(ga, outref(j), wsa)
        pltpu.make_async_copy(table_sh.at[idx_v.at[j + 1]], gb, gsb).wait()
        wa.wait()
        pltpu.async_copy(table_sh.at[idx_v.at[j + 2]], ga, gsa)
        pltpu.async_copy(gb, outref(j + 1), wsb)
        return carry

    lax.fori_loop(0, (NCH - 1) // 2, body, 0)
    pltpu.make_async_copy(table_sh.at[idx_v.at[NCH - 1]], ga, gsa).wait()
    pltpu.make_async_copy(gb, outref(NCH - 2), wsb).wait()
    pltpu.sync_copy(ga, outref(NCH - 1))


_BR = 2000


def _mlp_body(x_ref, g_ref, w1_ref, w1e_ref, w1o_ref, w2_ref, out_ref):
    gi = lax.bitcast_convert_type(g_ref[...], jnp.int32)
    ge = lax.bitcast_convert_type(gi << 16, jnp.float32)
    go = lax.bitcast_convert_type(gi & jnp.int32(-65536), jnp.float32)
    acc = lax.dot_general(
        x_ref[...], w1_ref[...], (((1,), (1,)), ((), ())),
        preferred_element_type=jnp.float32,
    )
    acc += lax.dot_general(
        ge, w1e_ref[...], (((1,), (1,)), ((), ())),
        preferred_element_type=jnp.float32,
    )
    acc += lax.dot_general(
        go, w1o_ref[...], (((1,), (1,)), ((), ())),
        preferred_element_type=jnp.float32,
    )
    h = jnp.maximum(acc, 0.0)
    out_ref[...] = lax.dot_general(
        h, w2_ref[...], (((1,), (1,)), ((), ())),
        preferred_element_type=jnp.float32,
    )


_mlp_call = pl.pallas_call(
    _mlp_body,
    grid=(N // _BR,),
    in_specs=[
        pl.BlockSpec((_BR, D), lambda i: (i, 0)),
        pl.BlockSpec((_BR, DP), lambda i: (i, 0)),
        pl.BlockSpec((D, D), lambda i: (0, 0)),
        pl.BlockSpec((D, DP), lambda i: (0, 0)),
        pl.BlockSpec((D, DP), lambda i: (0, 0)),
        pl.BlockSpec((D, D), lambda i: (0, 0)),
    ],
    out_specs=pl.BlockSpec((_BR, D), lambda i: (i, 0)),
    out_shape=jax.ShapeDtypeStruct((N, D), jnp.float32),
)


def kernel(x, edge_index, edge_attr, batch, W_lin, gamma, beta, W1, W2):
    del edge_index, edge_attr
    batch3 = batch.reshape(NW, NCH, CH)
    zeros = jnp.zeros((SEG_SLICE, D), jnp.float32)
    partials = _segment_sum_sc(x, batch3, zeros)
    table = _bn_call(partials, W_lin, gamma.reshape(1, D), beta.reshape(1, D))
    table_packed = lax.bitcast_convert_type(
        table.reshape(NSEG_PAD, DP, 2), jnp.float32
    )
    g = _gather_sc(table_packed, batch3)
    return _mlp_call(x, g, W1, W1[:, 0::2], W1[:, 1::2], W2)


# speedup vs baseline: 1.0184x; 1.0184x over previous
import functools

import jax
import jax.numpy as jnp
from jax import lax
from jax.experimental import pallas as pl
from jax.experimental.pallas import tpu as pltpu
from jax.experimental.pallas import tpu_sc as plsc

N = 320000
NSEG = 10000
NSEG_PAD = 10240
D = 128
BN_EPS = 1e-5

NC = 2
NS = 16
NW = NC * NS
ROWS_PER_W = N // NW
CH = 80
NCH = ROWS_PER_W // CH
SEG_SLICE = NSEG_PAD // NS
NSUB = SEG_SLICE // CH

_mesh = plsc.VectorSubcoreMesh(core_axis_name="c", subcore_axis_name="s")


@functools.partial(
    pl.kernel,
    out_type=jax.ShapeDtypeStruct((NC, NSEG_PAD, D), jnp.float32),
    mesh=_mesh,
    scratch_types=[
        pltpu.VMEM((NCH, CH), jnp.int32),
        pltpu.VMEM((CH, D), jnp.float32),
        pltpu.VMEM((CH, D), jnp.float32),
        pltpu.VMEM((CH, D), jnp.float32),
        pltpu.VMEM_SHARED((NSEG_PAD, D), jnp.float32),
        pltpu.SemaphoreType.DMA,
        pltpu.SemaphoreType.DMA,
        pltpu.SemaphoreType.DMA,
        pltpu.SemaphoreType.DMA,
        pltpu.SemaphoreType.DMA,
        pltpu.SemaphoreType.DMA,
    ],
)
def _segment_sum_sc(x_hbm, batch_hbm, zeros_hbm, out_hbm,
                    idx_v, x0, x1, x2, table_sh,
                    gs0, gs1, gs2, ss0, ss1, ss2):
    c = lax.axis_index("c")
    s = lax.axis_index("s")
    wid = c * NS + s
    base = wid * ROWS_PER_W
    bufs = (x0, x1, x2)
    gsems = (gs0, gs1, gs2)
    ssems = (ss0, ss1, ss2)

    def chunk(j):
        return x_hbm.at[pl.ds(base + j * CH, CH)]

    pltpu.sync_copy(batch_hbm.at[wid], idx_v)
    pltpu.sync_copy(zeros_hbm, table_sh.at[pl.ds(s * SEG_SLICE, SEG_SLICE)])
    plsc.subcore_barrier()

    pltpu.async_copy(chunk(0), bufs[0], gsems[0])
    pltpu.async_copy(chunk(1), bufs[1], gsems[1])

    def body(j3, carry):
        for k in range(3):
            j = 3 * j3 + k
            k2 = (k + 2) % 3
            pltpu.make_async_copy(chunk(j), bufs[k], gsems[k]).wait()
            pltpu.async_copy(bufs[k], table_sh.at[idx_v.at[j]], ssems[k],
                             add=True)
            if k == 0:
                @pl.when(j3 > 0)
                def _():
                    pltpu.make_async_copy(
                        bufs[k2], table_sh.at[idx_v.at[j]], ssems[k2]).wait()
            else:
                pltpu.make_async_copy(
                    bufs[k2], table_sh.at[idx_v.at[j]], ssems[k2]).wait()
            pltpu.async_copy(chunk(j + 2), bufs[k2], gsems[k2])
        return carry

    lax.fori_loop(0, (NCH - 2) // 3, body, 0)
    pltpu.make_async_copy(chunk(NCH - 2), bufs[0], gsems[0]).wait()
    pltpu.make_async_copy(bufs[2], table_sh.at[idx_v.at[NCH - 3]], ssems[2]).wait()
    sc123 = pltpu.async_copy(bufs[0], table_sh.at[idx_v.at[NCH - 2]], ssems[0],
                             add=True)
    pltpu.make_async_copy(chunk(NCH - 1), bufs[1], gsems[1]).wait()
    sc123.wait()
    pltpu.sync_copy(bufs[1], table_sh.at[idx_v.at[NCH - 1]], add=True)
    plsc.subcore_barrier()
    pltpu.sync_copy(
        table_sh.at[pl.ds(s * SEG_SLICE, SEG_SLICE)],
        out_hbm.at[c, pl.ds(s * SEG_SLICE, SEG_SLICE)],
    )


@functools.partial(
    pl.kernel,
    out_type=jax.ShapeDtypeStruct((N, D), jnp.float32),
    mesh=_mesh,
    scratch_types=[
        pltpu.VMEM((NCH, CH), jnp.int32),
        pltpu.VMEM((NSUB, CH), jnp.int32),
        pltpu.VMEM((CH, D), jnp.float32),
        pltpu.VMEM((CH, D), jnp.float32),
        pltpu.VMEM_SHARED((NSEG_PAD, D), jnp.float32),
        pltpu.SemaphoreType.DMA,
        pltpu.SemaphoreType.DMA,
        pltpu.SemaphoreType.DMA,
        pltpu.SemaphoreType.DMA,
    ],
)
def _gather_sc(partials_hbm, batch_hbm, iota_hbm, out_hbm,
               idx_v, iota_v, ga, gb, table_sh, gsa, gsb, wsa, wsb):
    c = lax.axis_index("c")
    s = lax.axis_index("s")
    wid = c * NS + s
    base = wid * ROWS_PER_W

    def outref(j):
        return out_hbm.at[pl.ds(base + j * CH, CH)]

    pltpu.sync_copy(
        partials_hbm.at[0, pl.ds(s * SEG_SLICE, SEG_SLICE)],
        table_sh.at[pl.ds(s * SEG_SLICE, SEG_SLICE)],
    )
    pltpu.sync_copy(iota_hbm.at[s], iota_v)
    pltpu.sync_copy(batch_hbm.at[wid], idx_v)

    def stage_body(u, carry):
        pltpu.sync_copy(
            partials_hbm.at[1, pl.ds(s * SEG_SLICE + u * CH, CH)], ga)
        pltpu.sync_copy(ga, table_sh.at[iota_v.at[u]], add=True)
        return carry

    lax.fori_loop(0, NSUB, stage_body, 0)
    plsc.subcore_barrier()

    pltpu.async_copy(table_sh.at[idx_v.at[0]], ga, gsa)

    def body(j2, carry):
        j = 2 * j2
        pltpu.make_async_copy(table_sh.at[idx_v.at[j]], ga, gsa).wait()

        @pl.when(j2 > 0)
        def _():
            pltpu.make_async_copy(gb, outref(j - 1), wsb).wait()

        pltpu.async_copy(table_sh.at[idx_v.at[j + 1]], gb, gsb)
        wa = pltpu.async_copy(ga, outref(j), wsa)
        pltpu.make_async_copy(table_sh.at[idx_v.at[j + 1]], gb, gsb).wait()
        wa.wait()
        pltpu.async_copy(table_sh.at[idx_v.at[j + 2]], ga, gsa)
        pltpu.async_copy(gb, outref(j + 1), wsb)
        return carry

    lax.fori_loop(0, (NCH - 1) // 2, body, 0)
    pltpu.make_async_copy(table_sh.at[idx_v.at[NCH - 1]], ga, gsa).wait()
    pltpu.make_async_copy(gb, outref(NCH - 2), wsb).wait()
    pltpu.sync_copy(ga, outref(NCH - 1))


_BR = 2000


def _mlp_body(x_ref, g_ref, p_ref, wlin_ref, gamma_ref, beta_ref,
              w1_ref, w2_ref, out_ref, s_ref, t_ref):
    @pl.when(pl.program_id(0) == 0)
    def _():
        summ = p_ref[0] + p_ref[1]
        z = lax.dot_general(
            summ, wlin_ref[...], (((1,), (1,)), ((), ())),
            preferred_element_type=jnp.float32,
        )
        mean = jnp.sum(z, axis=0, keepdims=True) / NSEG
        cent = z - mean
        ssq = jnp.sum(cent * cent, axis=0, keepdims=True) - (
            (NSEG_PAD - NSEG) * mean * mean
        )
        var = ssq / NSEG
        sc = gamma_ref[...] / jnp.sqrt(var + BN_EPS)
        s_ref[...] = sc
        t_ref[...] = beta_ref[...] - mean * sc

    zb = lax.dot_general(
        g_ref[...], wlin_ref[...], (((1,), (1,)), ((), ())),
        preferred_element_type=jnp.float32,
    )
    y = jnp.maximum(zb * s_ref[...] + t_ref[...], 0.0)
    h = x_ref[...] + y
    h = lax.dot_general(
        h, w1_ref[...], (((1,), (1,)), ((), ())),
        preferred_element_type=jnp.float32,
    )
    h = jnp.maximum(h, 0.0)
    out_ref[...] = lax.dot_general(
        h, w2_ref[...], (((1,), (1,)), ((), ())),
        preferred_element_type=jnp.float32,
    )


_mlp_call = pl.pallas_call(
    _mlp_body,
    grid=(N // _BR,),
    in_specs=[
        pl.BlockSpec((_BR, D), lambda i: (i, 0)),
        pl.BlockSpec((_BR, D), lambda i: (i, 0)),
        pl.BlockSpec((NC, NSEG_PAD, D), lambda i: (0, 0, 0)),
        pl.BlockSpec((D, D), lambda i: (0, 0)),
        pl.BlockSpec((1, D), lambda i: (0, 0)),
        pl.BlockSpec((1, D), lambda i: (0, 0)),
        pl.BlockSpec((D, D), lambda i: (0, 0)),
        pl.BlockSpec((D, D), lambda i: (0, 0)),
    ],
    out_specs=pl.BlockSpec((_BR, D), lambda i: (i, 0)),
    out_shape=jax.ShapeDtypeStruct((N, D), jnp.float32),
    scratch_shapes=[
        pltpu.VMEM((1, D), jnp.float32),
        pltpu.VMEM((1, D), jnp.float32),
    ],
)


def kernel(x, edge_index, edge_attr, batch, W_lin, gamma, beta, W1, W2):
    del edge_index, edge_attr
    batch3 = batch.reshape(NW, NCH, CH)
    zeros = jnp.zeros((SEG_SLICE, D), jnp.float32)
    iota3 = jnp.arange(NSEG_PAD, dtype=jnp.int32).reshape(NS, NSUB, CH)
    partials = _segment_sum_sc(x, batch3, zeros)
    graw = _gather_sc(partials, batch3, iota3)
    return _mlp_call(x, graw, partials, W_lin, gamma.reshape(1, D),
                     beta.reshape(1, D), W1, W2)


# speedup vs baseline: 1.2128x; 1.1909x over previous
import functools

import jax
import jax.numpy as jnp
from jax import lax
from jax.experimental import pallas as pl
from jax.experimental.pallas import tpu as pltpu
from jax.experimental.pallas import tpu_sc as plsc

N = 320000
NSEG = 10000
NSEG_PAD = 10240
D = 128
BN_EPS = 1e-5

NC = 2
NS = 16
NW = NC * NS
ROWS_PER_W = N // NW
CH = 80
NCH = ROWS_PER_W // CH
SEG_SLICE = NSEG_PAD // NS

_mesh = plsc.VectorSubcoreMesh(core_axis_name="c", subcore_axis_name="s")


@functools.partial(
    pl.kernel,
    out_type=jax.ShapeDtypeStruct((NC, NSEG_PAD, D), jnp.float32),
    mesh=_mesh,
    scratch_types=[
        pltpu.VMEM((NCH, CH), jnp.int32),
        pltpu.VMEM((CH, D), jnp.float32),
        pltpu.VMEM((CH, D), jnp.float32),
        pltpu.VMEM((CH, D), jnp.float32),
        pltpu.VMEM_SHARED((NSEG_PAD, D), jnp.float32),
        pltpu.SemaphoreType.DMA,
        pltpu.SemaphoreType.DMA,
        pltpu.SemaphoreType.DMA,
        pltpu.SemaphoreType.DMA,
        pltpu.SemaphoreType.DMA,
        pltpu.SemaphoreType.DMA,
    ],
)
def _segment_sum_sc(x_hbm, batch_hbm, zeros_hbm, out_hbm,
                    idx_v, x0, x1, x2, table_sh,
                    gs0, gs1, gs2, ss0, ss1, ss2):
    c = lax.axis_index("c")
    s = lax.axis_index("s")
    wid = c * NS + s
    base = wid * ROWS_PER_W
    bufs = (x0, x1, x2)
    gsems = (gs0, gs1, gs2)
    ssems = (ss0, ss1, ss2)

    def chunk(j):
        return x_hbm.at[pl.ds(base + j * CH, CH)]

    pltpu.sync_copy(batch_hbm.at[wid], idx_v)
    pltpu.sync_copy(zeros_hbm, table_sh.at[pl.ds(s * SEG_SLICE, SEG_SLICE)])
    plsc.subcore_barrier()

    pltpu.async_copy(chunk(0), bufs[0], gsems[0])
    pltpu.async_copy(chunk(1), bufs[1], gsems[1])

    def body(j3, carry):
        for k in range(3):
            j = 3 * j3 + k
            k2 = (k + 2) % 3
            pltpu.make_async_copy(chunk(j), bufs[k], gsems[k]).wait()
            pltpu.async_copy(bufs[k], table_sh.at[idx_v.at[j]], ssems[k],
                             add=True)
            if k == 0:
                @pl.when(j3 > 0)
                def _():
                    pltpu.make_async_copy(
                        bufs[k2], table_sh.at[idx_v.at[j]], ssems[k2]).wait()
            else:
                pltpu.make_async_copy(
                    bufs[k2], table_sh.at[idx_v.at[j]], ssems[k2]).wait()
            pltpu.async_copy(chunk(j + 2), bufs[k2], gsems[k2])
        return carry

    lax.fori_loop(0, (NCH - 2) // 3, body, 0)
    pltpu.make_async_copy(chunk(NCH - 2), bufs[0], gsems[0]).wait()
    pltpu.make_async_copy(bufs[2], table_sh.at[idx_v.at[NCH - 3]], ssems[2]).wait()
    sc123 = pltpu.async_copy(bufs[0], table_sh.at[idx_v.at[NCH - 2]], ssems[0],
                             add=True)
    pltpu.make_async_copy(chunk(NCH - 1), bufs[1], gsems[1]).wait()
    sc123.wait()
    pltpu.sync_copy(bufs[1], table_sh.at[idx_v.at[NCH - 1]], add=True)
    plsc.subcore_barrier()
    pltpu.sync_copy(
        table_sh.at[pl.ds(s * SEG_SLICE, SEG_SLICE)],
        out_hbm.at[c, pl.ds(s * SEG_SLICE, SEG_SLICE)],
    )


def _bn_body(p_ref, wlin_ref, gamma_ref, beta_ref, out_ref):
    summ = p_ref[0] + p_ref[1]
    summ = lax.dot_general(
        summ, wlin_ref[...], (((1,), (1,)), ((), ())),
        preferred_element_type=jnp.float32,
    )
    mean = jnp.sum(summ, axis=0, keepdims=True) / NSEG
    cent = summ - mean
    ssq = jnp.sum(cent * cent, axis=0, keepdims=True) - (
        (NSEG_PAD - NSEG) * mean * mean
    )
    var = ssq / NSEG
    y = cent / jnp.sqrt(var + BN_EPS) * gamma_ref[...] + beta_ref[...]
    out_ref[...] = jnp.maximum(y, 0.0)


_bn_call = pl.pallas_call(
    _bn_body,
    out_shape=jax.ShapeDtypeStruct((NSEG_PAD, D), jnp.float32),
)


@functools.partial(
    pl.kernel,
    out_type=jax.ShapeDtypeStruct((N, D), jnp.float32),
    mesh=_mesh,
    scratch_types=[
        pltpu.VMEM((NCH, CH), jnp.int32),
        pltpu.VMEM((CH, D), jnp.float32),
        pltpu.VMEM((CH, D), jnp.float32),
        pltpu.VMEM_SHARED((NSEG_PAD, D), jnp.float32),
        pltpu.SemaphoreType.DMA,
        pltpu.SemaphoreType.DMA,
        pltpu.SemaphoreType.DMA,
        pltpu.SemaphoreType.DMA,
    ],
)
def _gather_sc(table_hbm, batch_hbm, out_hbm,
               idx_v, ga, gb, table_sh, gsa, gsb, wsa, wsb):
    c = lax.axis_index("c")
    s = lax.axis_index("s")
    wid = c * NS + s
    base = wid * ROWS_PER_W

    def outref(j):
        return out_hbm.at[pl.ds(base + j * CH, CH)]

    pltpu.sync_copy(
        table_hbm.at[pl.ds(s * SEG_SLICE, SEG_SLICE)],
        table_sh.at[pl.ds(s * SEG_SLICE, SEG_SLICE)],
    )
    pltpu.sync_copy(batch_hbm.at[wid], idx_v)
    plsc.subcore_barrier()

    pltpu.async_copy(table_sh.at[idx_v.at[0]], ga, gsa)

    def body(j2, carry):
        j = 2 * j2
        pltpu.make_async_copy(table_sh.at[idx_v.at[j]], ga, gsa).wait()

        @pl.when(j2 > 0)
        def _():
            pltpu.make_async_copy(gb, outref(j - 1), wsb).wait()

        pltpu.async_copy(table_sh.at[idx_v.at[j + 1]], gb, gsb)
        wa = pltpu.async_copy(ga, outref(j), wsa)
        pltpu.make_async_copy(table_sh.at[idx_v.at[j + 1]], gb, gsb).wait()
        wa.wait()
        pltpu.async_copy(table_sh.at[idx_v.at[j + 2]], ga, gsa)
        pltpu.async_copy(gb, outref(j + 1), wsb)
        return carry

    lax.fori_loop(0, (NCH - 1) // 2, body, 0)
    pltpu.make_async_copy(table_sh.at[idx_v.at[NCH - 1]], ga, gsa).wait()
    pltpu.make_async_copy(gb, outref(NCH - 2), wsb).wait()
    pltpu.sync_copy(ga, outref(NCH - 1))


_BR = 4000


def _mlp_body(x_ref, g_ref, w1_ref, w2_ref, out_ref):
    h = x_ref[...] + g_ref[...]
    h = lax.dot_general(
        h, w1_ref[...], (((1,), (1,)), ((), ())),
        preferred_element_type=jnp.float32,
    )
    h = jnp.maximum(h, 0.0)
    out_ref[...] = lax.dot_general(
        h, w2_ref[...], (((1,), (1,)), ((), ())),
        preferred_element_type=jnp.float32,
    )


_mlp_call = pl.pallas_call(
    _mlp_body,
    grid=(N // _BR,),
    in_specs=[
        pl.BlockSpec((_BR, D), lambda i: (i, 0)),
        pl.BlockSpec((_BR, D), lambda i: (i, 0)),
        pl.BlockSpec((D, D), lambda i: (0, 0)),
        pl.BlockSpec((D, D), lambda i: (0, 0)),
    ],
    out_specs=pl.BlockSpec((_BR, D), lambda i: (i, 0)),
    out_shape=jax.ShapeDtypeStruct((N, D), jnp.float32),
)


def kernel(x, edge_index, edge_attr, batch, W_lin, gamma, beta, W1, W2):
    del edge_index, edge_attr
    batch3 = batch.reshape(NW, NCH, CH)
    zeros = jnp.zeros((SEG_SLICE, D), jnp.float32)
    partials = _segment_sum_sc(x, batch3, zeros)
    table = _bn_call(partials, W_lin, gamma.reshape(1, D), beta.reshape(1, D))
    g = _gather_sc(table, batch3)
    return _mlp_call(x, g, W1, W2)


# speedup vs baseline: 1.2494x; 1.0302x over previous
import functools

import jax
import jax.numpy as jnp
from jax import lax
from jax.experimental import pallas as pl
from jax.experimental.pallas import tpu as pltpu
from jax.experimental.pallas import tpu_sc as plsc

N = 320000
NSEG = 10000
NSEG_PAD = 10240
D = 128
BN_EPS = 1e-5

NC = 2
NS = 16
NW = NC * NS
ROWS_PER_W = N // NW
CH = 80
NCH = ROWS_PER_W // CH
SEG_SLICE = NSEG_PAD // NS

_mesh = plsc.VectorSubcoreMesh(core_axis_name="c", subcore_axis_name="s")


@functools.partial(
    pl.kernel,
    out_type=jax.ShapeDtypeStruct((NC, NSEG_PAD, D), jnp.float32),
    mesh=_mesh,
    scratch_types=[
        pltpu.VMEM((NCH, CH), jnp.int32),
        pltpu.VMEM((CH, D), jnp.float32),
        pltpu.VMEM((CH, D), jnp.float32),
        pltpu.VMEM((CH, D), jnp.float32),
        pltpu.VMEM_SHARED((NSEG_PAD, D), jnp.float32),
        pltpu.SemaphoreType.DMA,
        pltpu.SemaphoreType.DMA,
        pltpu.SemaphoreType.DMA,
        pltpu.SemaphoreType.DMA,
        pltpu.SemaphoreType.DMA,
        pltpu.SemaphoreType.DMA,
    ],
)
def _segment_sum_sc(x_hbm, batch_hbm, zeros_hbm, out_hbm,
                    idx_v, x0, x1, x2, table_sh,
                    gs0, gs1, gs2, ss0, ss1, ss2):
    c = lax.axis_index("c")
    s = lax.axis_index("s")
    wid = c * NS + s
    base = wid * ROWS_PER_W
    bufs = (x0, x1, x2)
    gsems = (gs0, gs1, gs2)
    ssems = (ss0, ss1, ss2)

    def chunk(j):
        return x_hbm.at[pl.ds(base + j * CH, CH)]

    pltpu.sync_copy(batch_hbm.at[wid], idx_v)
    pltpu.sync_copy(zeros_hbm, table_sh.at[pl.ds(s * SEG_SLICE, SEG_SLICE)])
    plsc.subcore_barrier()

    pltpu.async_copy(chunk(0), bufs[0], gsems[0])
    pltpu.async_copy(chunk(1), bufs[1], gsems[1])

    def body(j3, carry):
        for k in range(3):
            j = 3 * j3 + k
            k2 = (k + 2) % 3
            pltpu.make_async_copy(chunk(j), bufs[k], gsems[k]).wait()
            pltpu.async_copy(bufs[k], table_sh.at[idx_v.at[j]], ssems[k],
                             add=True)
            if k == 0:
                @pl.when(j3 > 0)
                def _():
                    pltpu.make_async_copy(
                        bufs[k2], table_sh.at[idx_v.at[j]], ssems[k2]).wait()
            else:
                pltpu.make_async_copy(
                    bufs[k2], table_sh.at[idx_v.at[j]], ssems[k2]).wait()
            pltpu.async_copy(chunk(j + 2), bufs[k2], gsems[k2])
        return carry

    lax.fori_loop(0, (NCH - 2) // 3, body, 0)
    pltpu.make_async_copy(chunk(NCH - 2), bufs[0], gsems[0]).wait()
    pltpu.make_async_copy(bufs[2], table_sh.at[idx_v.at[NCH - 3]], ssems[2]).wait()
    sc123 = pltpu.async_copy(bufs[0], table_sh.at[idx_v.at[NCH - 2]], ssems[0],
                             add=True)
    pltpu.make_async_copy(chunk(NCH - 1), bufs[1], gsems[1]).wait()
    sc123.wait()
    pltpu.sync_copy(bufs[1], table_sh.at[idx_v.at[NCH - 1]], add=True)
    plsc.subcore_barrier()
    pltpu.sync_copy(
        table_sh.at[pl.ds(s * SEG_SLICE, SEG_SLICE)],
        out_hbm.at[c, pl.ds(s * SEG_SLICE, SEG_SLICE)],
    )


def _bn_body(p_ref, wlin_ref, gamma_ref, beta_ref, out_ref):
    summ = p_ref[0] + p_ref[1]
    summ = lax.dot_general(
        summ, wlin_ref[...], (((1,), (1,)), ((), ())),
        preferred_element_type=jnp.float32,
    )
    mean = jnp.sum(summ, axis=0, keepdims=True) / NSEG
    cent = summ - mean
    ssq = jnp.sum(cent * cent, axis=0, keepdims=True) - (
        (NSEG_PAD - NSEG) * mean * mean
    )
    var = ssq / NSEG
    y = cent / jnp.sqrt(var + BN_EPS) * gamma_ref[...] + beta_ref[...]
    out_ref[...] = jnp.maximum(y, 0.0)


_bn_call = pl.pallas_call(
    _bn_body,
    out_shape=jax.ShapeDtypeStruct((NSEG_PAD, D), jnp.float32),
)


@functools.partial(
    pl.kernel,
    out_type=jax.ShapeDtypeStruct((N, D), jnp.float32),
    mesh=_mesh,
    scratch_types=[
        pltpu.VMEM((NCH, CH), jnp.int32),
        pltpu.VMEM((CH, D), jnp.float32),
        pltpu.VMEM((CH, D), jnp.float32),
        pltpu.VMEM_SHARED((NSEG_PAD, D), jnp.float32),
        pltpu.SemaphoreType.DMA,
        pltpu.SemaphoreType.DMA,
        pltpu.SemaphoreType.DMA,
        pltpu.SemaphoreType.DMA,
    ],
)
def _gather_sc(table_hbm, batch_hbm, out_hbm,
               idx_v, ga, gb, table_sh, gsa, gsb, wsa, wsb):
    c = lax.axis_index("c")
    s = lax.axis_index("s")
    wid = c * NS + s
    base = wid * ROWS_PER_W

    def outref(j):
        return out_hbm.at[pl.ds(base + j * CH, CH)]

    pltpu.sync_copy(
        table_hbm.at[pl.ds(s * SEG_SLICE, SEG_SLICE)],
        table_sh.at[pl.ds(s * SEG_SLICE, SEG_SLICE)],
    )
    pltpu.sync_copy(batch_hbm.at[wid], idx_v)
    plsc.subcore_barrier()

    pltpu.async_copy(table_sh.at[idx_v.at[0]], ga, gsa)

    def body(j2, carry):
        j = 2 * j2
        pltpu.make_async_copy(table_sh.at[idx_v.at[j]], ga, gsa).wait()

        @pl.when(j2 > 0)
        def _():
            pltpu.make_async_copy(gb, outref(j - 1), wsb).wait()

        pltpu.async_copy(table_sh.at[idx_v.at[j + 1]], gb, gsb)
        wa = pltpu.async_copy(ga, outref(j), wsa)
        pltpu.make_async_copy(table_sh.at[idx_v.at[j + 1]], gb, gsb).wait()
        wa.wait()
        pltpu.async_copy(table_sh.at[idx_v.at[j + 2]], ga, gsa)
        pltpu.async_copy(gb, outref(j + 1), wsb)
        return carry

    lax.fori_loop(0, (NCH - 1) // 2, body, 0)
    pltpu.make_async_copy(table_sh.at[idx_v.at[NCH - 1]], ga, gsa).wait()
    pltpu.make_async_copy(gb, outref(NCH - 2), wsb).wait()
    pltpu.sync_copy(ga, outref(NCH - 1))


_BR = 8000


def _mlp_body(x_ref, g_ref, w1_ref, w2_ref, out_ref):
    h = x_ref[...] + g_ref[...]
    h = lax.dot_general(
        h, w1_ref[...], (((1,), (1,)), ((), ())),
        preferred_element_type=jnp.float32,
    )
    h = jnp.maximum(h, 0.0)
    out_ref[...] = lax.dot_general(
        h, w2_ref[...], (((1,), (1,)), ((), ())),
        preferred_element_type=jnp.float32,
    )


_mlp_call = pl.pallas_call(
    _mlp_body,
    grid=(N // _BR,),
    in_specs=[
        pl.BlockSpec((_BR, D), lambda i: (i, 0)),
        pl.BlockSpec((_BR, D), lambda i: (i, 0)),
        pl.BlockSpec((D, D), lambda i: (0, 0)),
        pl.BlockSpec((D, D), lambda i: (0, 0)),
    ],
    out_specs=pl.BlockSpec((_BR, D), lambda i: (i, 0)),
    out_shape=jax.ShapeDtypeStruct((N, D), jnp.float32),
)


def kernel(x, edge_index, edge_attr, batch, W_lin, gamma, beta, W1, W2):
    del edge_index, edge_attr
    batch3 = batch.reshape(NW, NCH, CH)
    zeros = jnp.zeros((SEG_SLICE, D), jnp.float32)
    partials = _segment_sum_sc(x, batch3, zeros)
    table = _bn_call(partials, W_lin, gamma.reshape(1, D), beta.reshape(1, D))
    g = _gather_sc(table, batch3)
    return _mlp_call(x, g, W1, W2)


# speedup vs baseline: 1.2849x; 1.0284x over previous
import functools

import jax
import jax.numpy as jnp
from jax import lax
from jax.experimental import pallas as pl
from jax.experimental.pallas import tpu as pltpu
from jax.experimental.pallas import tpu_sc as plsc

N = 320000
NSEG = 10000
NSEG_PAD = 10240
D = 128
BN_EPS = 1e-5

NC = 2
NS = 16
NW = NC * NS
ROWS_PER_W = N // NW
CH = 80
NCH = ROWS_PER_W // CH
SEG_SLICE = NSEG_PAD // NS

_mesh = plsc.VectorSubcoreMesh(core_axis_name="c", subcore_axis_name="s")


@functools.partial(
    pl.kernel,
    out_type=jax.ShapeDtypeStruct((NC, NSEG_PAD, D), jnp.float32),
    mesh=_mesh,
    scratch_types=[
        pltpu.VMEM((NCH, CH), jnp.int32),
        pltpu.VMEM((CH, D), jnp.float32),
        pltpu.VMEM((CH, D), jnp.float32),
        pltpu.VMEM((CH, D), jnp.float32),
        pltpu.VMEM_SHARED((NSEG_PAD, D), jnp.float32),
        pltpu.SemaphoreType.DMA,
        pltpu.SemaphoreType.DMA,
        pltpu.SemaphoreType.DMA,
        pltpu.SemaphoreType.DMA,
        pltpu.SemaphoreType.DMA,
        pltpu.SemaphoreType.DMA,
    ],
)
def _segment_sum_sc(x_hbm, batch_hbm, zeros_hbm, out_hbm,
                    idx_v, x0, x1, x2, table_sh,
                    gs0, gs1, gs2, ss0, ss1, ss2):
    c = lax.axis_index("c")
    s = lax.axis_index("s")
    wid = c * NS + s
    base = wid * ROWS_PER_W
    bufs = (x0, x1, x2)
    gsems = (gs0, gs1, gs2)
    ssems = (ss0, ss1, ss2)

    def chunk(j):
        return x_hbm.at[pl.ds(base + j * CH, CH)]

    pltpu.sync_copy(batch_hbm.at[wid], idx_v)
    pltpu.sync_copy(zeros_hbm, table_sh.at[pl.ds(s * SEG_SLICE, SEG_SLICE)])
    plsc.subcore_barrier()

    pltpu.async_copy(chunk(0), bufs[0], gsems[0])
    pltpu.async_copy(chunk(1), bufs[1], gsems[1])

    def body(j3, carry):
        for k in range(3):
            j = 3 * j3 + k
            k2 = (k + 2) % 3
            pltpu.make_async_copy(chunk(j), bufs[k], gsems[k]).wait()
            pltpu.async_copy(bufs[k], table_sh.at[idx_v.at[j]], ssems[k],
                             add=True)
            if k == 0:
                @pl.when(j3 > 0)
                def _():
                    pltpu.make_async_copy(
                        bufs[k2], table_sh.at[idx_v.at[j]], ssems[k2]).wait()
            else:
                pltpu.make_async_copy(
                    bufs[k2], table_sh.at[idx_v.at[j]], ssems[k2]).wait()
            pltpu.async_copy(chunk(j + 2), bufs[k2], gsems[k2])
        return carry

    lax.fori_loop(0, (NCH - 2) // 3, body, 0)
    pltpu.make_async_copy(chunk(NCH - 2), bufs[0], gsems[0]).wait()
    pltpu.make_async_copy(bufs[2], table_sh.at[idx_v.at[NCH - 3]], ssems[2]).wait()
    sc123 = pltpu.async_copy(bufs[0], table_sh.at[idx_v.at[NCH - 2]], ssems[0],
                             add=True)
    pltpu.make_async_copy(chunk(NCH - 1), bufs[1], gsems[1]).wait()
    sc123.wait()
    pltpu.sync_copy(bufs[1], table_sh.at[idx_v.at[NCH - 1]], add=True)
    plsc.subcore_barrier()
    pltpu.sync_copy(
        table_sh.at[pl.ds(s * SEG_SLICE, SEG_SLICE)],
        out_hbm.at[c, pl.ds(s * SEG_SLICE, SEG_SLICE)],
    )


def _bn_body(p_ref, wlin_ref, gamma_ref, beta_ref, out_ref):
    summ = p_ref[0] + p_ref[1]
    summ = lax.dot_general(
        summ, wlin_ref[...], (((1,), (1,)), ((), ())),
        preferred_element_type=jnp.float32,
    )
    mean = jnp.sum(summ, axis=0, keepdims=True) / NSEG
    cent = summ - mean
    ssq = jnp.sum(cent * cent, axis=0, keepdims=True) - (
        (NSEG_PAD - NSEG) * mean * mean
    )
    var = ssq / NSEG
    y = cent / jnp.sqrt(var + BN_EPS) * gamma_ref[...] + beta_ref[...]
    out_ref[...] = jnp.maximum(y, 0.0)


_bn_call = pl.pallas_call(
    _bn_body,
    out_shape=jax.ShapeDtypeStruct((NSEG_PAD, D), jnp.float32),
)


@functools.partial(
    pl.kernel,
    out_type=jax.ShapeDtypeStruct((N, D), jnp.float32),
    mesh=_mesh,
    scratch_types=[
        pltpu.VMEM((NCH, CH), jnp.int32),
        pltpu.VMEM((CH, D), jnp.float32),
        pltpu.VMEM((CH, D), jnp.float32),
        pltpu.VMEM((CH, D), jnp.float32),
        pltpu.VMEM_SHARED((NSEG_PAD, D), jnp.float32),
        pltpu.SemaphoreType.DMA,
        pltpu.SemaphoreType.DMA,
        pltpu.SemaphoreType.DMA,
        pltpu.SemaphoreType.DMA,
        pltpu.SemaphoreType.DMA,
        pltpu.SemaphoreType.DMA,
    ],
)
def _gather_sc(table_hbm, batch_hbm, out_hbm,
               idx_v, g0, g1, g2, table_sh, gsa0, gsa1, gsa2, ws0, ws1, ws2):
    c = lax.axis_index("c")
    s = lax.axis_index("s")
    wid = c * NS + s
    base = wid * ROWS_PER_W
    bufs = (g0, g1, g2)
    gsems = (gsa0, gsa1, gsa2)
    wsems = (ws0, ws1, ws2)

    def outref(j):
        return out_hbm.at[pl.ds(base + j * CH, CH)]

    pltpu.sync_copy(
        table_hbm.at[pl.ds(s * SEG_SLICE, SEG_SLICE)],
        table_sh.at[pl.ds(s * SEG_SLICE, SEG_SLICE)],
    )
    pltpu.sync_copy(batch_hbm.at[wid], idx_v)
    plsc.subcore_barrier()

    pltpu.async_copy(table_sh.at[idx_v.at[0]], bufs[0], gsems[0])
    pltpu.async_copy(table_sh.at[idx_v.at[1]], bufs[1], gsems[1])

    def body(j3, carry):
        for k in range(3):
            j = 3 * j3 + k
            k2 = (k + 2) % 3
            pltpu.make_async_copy(
                table_sh.at[idx_v.at[j]], bufs[k], gsems[k]).wait()
            pltpu.async_copy(bufs[k], outref(j), wsems[k])
            if k == 0:
                @pl.when(j3 > 0)
                def _():
                    pltpu.make_async_copy(bufs[k2], outref(j), wsems[k2]).wait()
            else:
                pltpu.make_async_copy(bufs[k2], outref(j), wsems[k2]).wait()
            pltpu.async_copy(table_sh.at[idx_v.at[j + 2]], bufs[k2], gsems[k2])
        return carry

    lax.fori_loop(0, (NCH - 2) // 3, body, 0)
    pltpu.make_async_copy(
        table_sh.at[idx_v.at[NCH - 2]], bufs[0], gsems[0]).wait()
    pltpu.make_async_copy(bufs[2], outref(NCH - 3), wsems[2]).wait()
    w123 = pltpu.async_copy(bufs[0], outref(NCH - 2), wsems[0])
    pltpu.make_async_copy(
        table_sh.at[idx_v.at[NCH - 1]], bufs[1], gsems[1]).wait()
    w123.wait()
    pltpu.sync_copy(bufs[1], outref(NCH - 1))


_BR = 10000


def _mlp_body(x_ref, g_ref, w1_ref, w2_ref, out_ref):
    h = x_ref[...] + g_ref[...]
    h = lax.dot_general(
        h, w1_ref[...], (((1,), (1,)), ((), ())),
        preferred_element_type=jnp.float32,
    )
    h = jnp.maximum(h, 0.0)
    out_ref[...] = lax.dot_general(
        h, w2_ref[...], (((1,), (1,)), ((), ())),
        preferred_element_type=jnp.float32,
    )


_mlp_call = pl.pallas_call(
    _mlp_body,
    grid=(N // _BR,),
    in_specs=[
        pl.BlockSpec((_BR, D), lambda i: (i, 0)),
        pl.BlockSpec((_BR, D), lambda i: (i, 0)),
        pl.BlockSpec((D, D), lambda i: (0, 0)),
        pl.BlockSpec((D, D), lambda i: (0, 0)),
    ],
    out_specs=pl.BlockSpec((_BR, D), lambda i: (i, 0)),
    out_shape=jax.ShapeDtypeStruct((N, D), jnp.float32),
)


def kernel(x, edge_index, edge_attr, batch, W_lin, gamma, beta, W1, W2):
    del edge_index, edge_attr
    batch3 = batch.reshape(NW, NCH, CH)
    zeros = jnp.zeros((SEG_SLICE, D), jnp.float32)
    partials = _segment_sum_sc(x, batch3, zeros)
    table = _bn_call(partials, W_lin, gamma.reshape(1, D), beta.reshape(1, D))
    g = _gather_sc(table, batch3)
    return _mlp_call(x, g, W1, W2)
